# baseline (speedup 1.0000x reference)
import functools

import jax
import jax.numpy as jnp
from jax import lax
from jax.experimental import pallas as pl
from jax.experimental.pallas import tpu as pltpu
from jax.experimental.pallas import tpu_sc as plsc

N = 10000
E = 320000
D = 128

NC = 2
NS = 16
NW = NC * NS
LANES = 16

N_PAD = 10240
BLK = 64
GRP = 4
SWBLK = 128
EPT = 10240
E_PAD = EPT * NW
NBLK = EPT // BLK
NGRP_T = NBLK // GRP
NGRP = E_PAD // (BLK * GRP)
SWNBLK = EPT // SWBLK
ROWS_PER_TILE = N_PAD // NS


def _sc_agg_body(h_hbm, pi_hbm, pw_hbm, agg_out,
                 agg_sh, rows_a, rows_b, pi_a, pw_a, pi_b, pw_b,
                 g_a, g_b, sem_pa, sem_pb):
  c = lax.axis_index("c")
  s = lax.axis_index("s")
  wid = c * NS + s
  rows = [rows_a, rows_b]
  G = [g_a, g_b]
  base_g = wid * NGRP_T
  max_g = NGRP - 1

  def zero_rows(i, _):
    for j in range(D // LANES):
      rows_a[i, pl.ds(j * LANES, LANES)] = jnp.zeros((LANES,), jnp.float32)
    return 0
  lax.fori_loop(0, BLK, zero_rows, 0)
  for k in range(ROWS_PER_TILE // BLK):
    pltpu.sync_copy(rows_a, agg_sh.at[pl.ds(s * ROWS_PER_TILE + k * BLK, BLK)])
  plsc.subcore_barrier()

  def p_start(goff, piv, pwv, sem):
    goff = jnp.minimum(goff, max_g)
    pltpu.async_copy(pi_hbm.at[goff], piv, sem)
    pltpu.async_copy(pw_hbm.at[goff], pwv, sem)

  def p_wait(goff, piv, pwv, sem):
    goff = jnp.minimum(goff, max_g)
    pltpu.make_async_copy(pi_hbm.at[goff], piv, sem).wait()
    pltpu.make_async_copy(pw_hbm.at[goff], pwv, sem).wait()

  def gather_start(piv, b, r):
    pltpu.async_copy(h_hbm.at[piv.at[2 * b]], rows[r], G[r])

  def gather_wait(piv, b, r):
    pltpu.make_async_copy(h_hbm.at[piv.at[2 * b]], rows[r], G[r]).wait()

  def scale(r, pwv, b):
    rr = rows[r]
    def scale_group(g, _):
      wv = pwv[b, pl.ds(g * LANES, LANES)]
      for k in range(LANES):
        we = wv[k]
        row = g * LANES + k
        for j in range(D // LANES):
          rr[row, pl.ds(j * LANES, LANES)] = (
              rr[row, pl.ds(j * LANES, LANES)] * we)
      return 0
    lax.fori_loop(0, BLK // LANES, scale_group, 0)

  p_start(base_g, pi_a, pw_a, sem_pa)
  p_wait(base_g, pi_a, pw_a, sem_pa)
  gather_start(pi_a, 0, 0)
  p_start(base_g + 1, pi_b, pw_b, sem_pb)

  def two_groups(q, _):
    g0 = base_g + 2 * q
    for b in range(8):
      bl = b % 4
      bln = (b + 1) % 4
      piv, pwv = (pi_a, pw_a) if b < 4 else (pi_b, pw_b)
      piv_next = pi_a if (b + 1) < 4 else (pi_b if b < 7 else pi_a)
      r = b % 2
      rn = (b + 1) % 2
      if b == 3:
        p_wait(g0 + 1, pi_b, pw_b, sem_pb)
      if b == 7:
        p_wait(g0 + 2, pi_a, pw_a, sem_pa)
      gather_wait(piv, bl, r)
      gather_start(piv_next, bln, rn)
      scale(r, pwv, bl)
      pltpu.sync_copy(rows[r], agg_sh.at[piv.at[2 * bl + 1]], add=True)
      if b == 3:
        p_start(g0 + 2, pi_a, pw_a, sem_pa)
      if b == 7:
        p_start(g0 + 3, pi_b, pw_b, sem_pb)
    return 0

  lax.fori_loop(0, NGRP_T // 2, two_groups, 0)
  gather_wait(pi_a, 0, 0)
  p_wait(base_g + NGRP_T + 1, pi_b, pw_b, sem_pb)
  plsc.subcore_barrier()

  pltpu.sync_copy(
      agg_sh.at[pl.ds(s * ROWS_PER_TILE, ROWS_PER_TILE)],
      agg_out.at[pl.ds(c * N_PAD + s * ROWS_PER_TILE, ROWS_PER_TILE)])


_sc_agg = pl.kernel(
    _sc_agg_body,
    out_type=jax.ShapeDtypeStruct((NC * N_PAD, D), jnp.float32),
    mesh=plsc.VectorSubcoreMesh(
        core_axis_name="c", subcore_axis_name="s",
        num_cores=NC, num_subcores=NS),
    scratch_types=[
        pltpu.VMEM_SHARED((N_PAD, D), jnp.float32),
        pltpu.VMEM((BLK, D), jnp.float32),
        pltpu.VMEM((BLK, D), jnp.float32),
        pltpu.VMEM((2 * GRP, BLK), jnp.int32),
        pltpu.VMEM((GRP, BLK), jnp.float32),
        pltpu.VMEM((2 * GRP, BLK), jnp.int32),
        pltpu.VMEM((GRP, BLK), jnp.float32),
        pltpu.SemaphoreType.DMA,
        pltpu.SemaphoreType.DMA,
        pltpu.SemaphoreType.DMA,
        pltpu.SemaphoreType.DMA,
    ],
)


def _sc_sw_body(dst_hbm, w_hbm, sw_out, acc_sh, wrep_v, dst_v, w_v):
  c = lax.axis_index("c")
  s = lax.axis_index("s")
  wid = c * NS + s

  def zero_rows(i, _):
    for j in range(D // LANES):
      wrep_v[i, pl.ds(j * LANES, LANES)] = jnp.zeros((LANES,), jnp.float32)
    return 0
  lax.fori_loop(0, SWBLK, zero_rows, 0)
  for k in range(ROWS_PER_TILE // SWBLK):
    pltpu.sync_copy(wrep_v,
                    acc_sh.at[pl.ds(s * ROWS_PER_TILE + k * SWBLK, SWBLK)])
  plsc.subcore_barrier()

  base_e = wid * EPT

  def edge_block(blk, _):
    off = base_e + blk * SWBLK
    pltpu.sync_copy(dst_hbm.at[pl.ds(off, SWBLK)], dst_v)
    pltpu.sync_copy(w_hbm.at[pl.ds(off, SWBLK)], w_v)

    def fill_group(g, _):
      wv = w_v[pl.ds(g * LANES, LANES)]
      for k in range(LANES):
        wrep_v[g * LANES + k, pl.ds(0, LANES)] = (
            jnp.broadcast_to(wv[k], (LANES,)))
      return 0
    lax.fori_loop(0, SWBLK // LANES, fill_group, 0)

    pltpu.sync_copy(wrep_v, acc_sh.at[dst_v], add=True)
    return 0

  lax.fori_loop(0, SWNBLK, edge_block, 0)
  plsc.subcore_barrier()
  pltpu.sync_copy(
      acc_sh.at[pl.ds(s * ROWS_PER_TILE, ROWS_PER_TILE)],
      sw_out.at[pl.ds(c * N_PAD + s * ROWS_PER_TILE, ROWS_PER_TILE)])


_sc_sw = pl.kernel(
    _sc_sw_body,
    out_type=jax.ShapeDtypeStruct((NC * N_PAD, D), jnp.float32),
    mesh=plsc.VectorSubcoreMesh(
        core_axis_name="c", subcore_axis_name="s",
        num_cores=NC, num_subcores=NS),
    scratch_types=[
        pltpu.VMEM_SHARED((N_PAD, D), jnp.float32),
        pltpu.VMEM((SWBLK, D), jnp.float32),
        pltpu.VMEM((SWBLK,), jnp.int32),
        pltpu.VMEM((SWBLK,), jnp.float32),
    ],
)


R_BLK = 512


def _dense_body(with_head, a0_ref, a1_ref, h_ref, sw0_ref, sw1_ref,
                wn_ref, wr_ref, br_ref, g_ref, b_ref, fcw_ref, fcb_ref,
                out_ref, logit_ref):
  sw = jnp.maximum(sw0_ref[:, 0:1] + sw1_ref[:, 0:1], 1.0)
  agg = (a0_ref[...] + a1_ref[...]) / sw
  h = h_ref[...]
  out = (jnp.dot(agg, wn_ref[...], preferred_element_type=jnp.float32,
                 precision=lax.Precision.HIGHEST)
         + jnp.dot(h, wr_ref[...], preferred_element_type=jnp.float32,
                   precision=lax.Precision.HIGHEST)
         + br_ref[...])
  out = jnp.maximum(out, 0.0)
  mu = jnp.mean(out, axis=-1, keepdims=True)
  cen = out - mu
  var = jnp.mean(cen * cen, axis=-1, keepdims=True)
  out = cen * lax.rsqrt(var + 1e-5) * g_ref[...] + b_ref[...]
  out_ref[...] = out
  if with_head:
    logit_ref[...] = jnp.dot(
        out, fcw_ref[...], preferred_element_type=jnp.float32,
        precision=lax.Precision.HIGHEST) + fcb_ref[...]


def _make_dense(with_head):
  grid = N_PAD // R_BLK
  row_spec = pl.BlockSpec((R_BLK, D), lambda i: (i, 0))
  full_spec = pl.BlockSpec((D, D), lambda i: (0, 0))
  vec_spec = pl.BlockSpec((1, D), lambda i: (0, 0))
  out_shapes = (jax.ShapeDtypeStruct((N_PAD, D), jnp.float32),
                jax.ShapeDtypeStruct((N_PAD, 1), jnp.float32))
  out_specs = (row_spec, pl.BlockSpec((R_BLK, 1), lambda i: (i, 0)))
  return pl.pallas_call(
      functools.partial(_dense_body, with_head),
      grid=(grid,),
      in_specs=[
          row_spec, row_spec, row_spec, row_spec, row_spec,
          full_spec, full_spec, vec_spec, vec_spec, vec_spec,
          pl.BlockSpec((D, 1), lambda i: (0, 0)),
          pl.BlockSpec((1, 1), lambda i: (0, 0)),
      ],
      out_specs=out_specs,
      out_shape=out_shapes,
  )


_dense_mid = _make_dense(False)
_dense_head = _make_dense(True)


def kernel(x, edge_index, edge_weight,
           W_neigh0, W_root0, b_root0, ln_g0, ln_b0,
           W_neigh1, W_root1, b_root1, ln_g1, ln_b1,
           W_neigh2, W_root2, b_root2, ln_g2, ln_b2,
           fc_w, fc_b):
  h = jnp.zeros((N_PAD, D), jnp.float32).at[:N].set(x)
  pad_e = E_PAD - E
  src = jnp.concatenate([edge_index[0], jnp.zeros((pad_e,), jnp.int32)])
  dst = jnp.concatenate([edge_index[1], jnp.zeros((pad_e,), jnp.int32)])
  w = jnp.concatenate([edge_weight, jnp.zeros((pad_e,), jnp.float32)])
  pi = jnp.stack([src.reshape(-1, BLK), dst.reshape(-1, BLK)],
                 axis=1).reshape(NGRP, 2 * GRP, BLK)
  pw = w.reshape(NGRP, GRP, BLK)

  sw_flat = _sc_sw(dst, w)
  sw0 = sw_flat[:N_PAD]
  sw1 = sw_flat[N_PAD:]

  layers = [
      (W_neigh0, W_root0, b_root0, ln_g0, ln_b0),
      (W_neigh1, W_root1, b_root1, ln_g1, ln_b1),
      (W_neigh2, W_root2, b_root2, ln_g2, ln_b2),
  ]
  logits = None
  for li, (Wn, Wr, br, g, b) in enumerate(layers):
    agg_flat = _sc_agg(h, pi, pw)
    a0 = agg_flat[:N_PAD]
    a1 = agg_flat[N_PAD:]
    dense = _dense_head if li == 2 else _dense_mid
    h, lg = dense(a0, a1, h, sw0, sw1, Wn.T, Wr.T, br.reshape(1, D),
                  g.reshape(1, D), b.reshape(1, D), fc_w.T,
                  fc_b.reshape(1, 1))
    if li == 2:
      logits = lg
  return (logits[:N, 0], h[:N])


# speedup vs baseline: 1.3665x; 1.3665x over previous
import functools

import jax
import jax.numpy as jnp
from jax import lax
from jax.experimental import pallas as pl
from jax.experimental.pallas import tpu as pltpu
from jax.experimental.pallas import tpu_sc as plsc

N = 10000
E = 320000
D = 128

NC = 2
NS = 16
NW = NC * NS
LANES = 16

N_PAD = 10240
BLK = 64
SWBLK = 128
EPT = ((E + NW * SWBLK - 1) // (NW * SWBLK)) * SWBLK
E_PAD = EPT * NW
NBLK = EPT // BLK
NPAIR = NBLK // 2
SWNBLK = EPT // SWBLK
ROWS_PER_TILE = N_PAD // NS
MAXOFF = E_PAD - BLK


def _sc_agg_body(h_hbm, src_hbm, dst_hbm, w_hbm, agg_out,
                 agg_sh, rows_a, rows_b, src_a, dst_a, w_a,
                 src_b, dst_b, w_b, sem_ia, sem_ib, sem_ga, sem_gb):
  c = lax.axis_index("c")
  s = lax.axis_index("s")
  wid = c * NS + s

  def zero_rows(i, _):
    for j in range(D // LANES):
      rows_a[i, pl.ds(j * LANES, LANES)] = jnp.zeros((LANES,), jnp.float32)
    return 0
  lax.fori_loop(0, BLK, zero_rows, 0)
  for k in range(ROWS_PER_TILE // BLK):
    pltpu.sync_copy(rows_a, agg_sh.at[pl.ds(s * ROWS_PER_TILE + k * BLK, BLK)])
  plsc.subcore_barrier()

  base_e = wid * EPT

  def idx_start(off, bufs, sem):
    sv, dv, wv = bufs
    off = jnp.minimum(off, MAXOFF)
    pltpu.async_copy(src_hbm.at[pl.ds(off, BLK)], sv, sem)
    pltpu.async_copy(dst_hbm.at[pl.ds(off, BLK)], dv, sem)
    pltpu.async_copy(w_hbm.at[pl.ds(off, BLK)], wv, sem)

  def idx_wait(off, bufs, sem):
    sv, dv, wv = bufs
    off = jnp.minimum(off, MAXOFF)
    pltpu.make_async_copy(src_hbm.at[pl.ds(off, BLK)], sv, sem).wait()
    pltpu.make_async_copy(dst_hbm.at[pl.ds(off, BLK)], dv, sem).wait()
    pltpu.make_async_copy(w_hbm.at[pl.ds(off, BLK)], wv, sem).wait()

  def scale(rows, wv_ref):
    def scale_group(g, _):
      wv = wv_ref[pl.ds(g * LANES, LANES)]
      for k in range(LANES):
        we = wv[k]
        row = g * LANES + k
        for j in range(D // LANES):
          rows[row, pl.ds(j * LANES, LANES)] = (
              rows[row, pl.ds(j * LANES, LANES)] * we)
      return 0
    lax.fori_loop(0, BLK // LANES, scale_group, 0)

  bufs_a = (src_a, dst_a, w_a)
  bufs_b = (src_b, dst_b, w_b)

  idx_start(base_e, bufs_a, sem_ia)
  idx_wait(base_e, bufs_a, sem_ia)
  pltpu.async_copy(h_hbm.at[src_a], rows_a, sem_ga)
  idx_start(base_e + BLK, bufs_b, sem_ib)

  def pair(p, _):
    off0 = base_e + (2 * p) * BLK
    pltpu.make_async_copy(h_hbm.at[src_a], rows_a, sem_ga).wait()
    idx_wait(off0 + BLK, bufs_b, sem_ib)
    pltpu.async_copy(h_hbm.at[src_b], rows_b, sem_gb)
    scale(rows_a, w_a)
    pltpu.sync_copy(rows_a, agg_sh.at[dst_a], add=True)
    idx_start(off0 + 2 * BLK, bufs_a, sem_ia)
    pltpu.make_async_copy(h_hbm.at[src_b], rows_b, sem_gb).wait()
    idx_wait(off0 + 2 * BLK, bufs_a, sem_ia)
    pltpu.async_copy(h_hbm.at[src_a], rows_a, sem_ga)
    scale(rows_b, w_b)
    pltpu.sync_copy(rows_b, agg_sh.at[dst_b], add=True)
    idx_start(off0 + 3 * BLK, bufs_b, sem_ib)
    return 0

  lax.fori_loop(0, NPAIR, pair, 0)
  pltpu.make_async_copy(h_hbm.at[src_a], rows_a, sem_ga).wait()
  idx_wait(base_e + NBLK * BLK, bufs_b, sem_ib)
  plsc.subcore_barrier()

  pltpu.sync_copy(
      agg_sh.at[pl.ds(s * ROWS_PER_TILE, ROWS_PER_TILE)],
      agg_out.at[pl.ds(c * N_PAD + s * ROWS_PER_TILE, ROWS_PER_TILE)])


_sc_agg = pl.kernel(
    _sc_agg_body,
    out_type=jax.ShapeDtypeStruct((NC * N_PAD, D), jnp.float32),
    mesh=plsc.VectorSubcoreMesh(
        core_axis_name="c", subcore_axis_name="s",
        num_cores=NC, num_subcores=NS),
    scratch_types=[
        pltpu.VMEM_SHARED((N_PAD, D), jnp.float32),
        pltpu.VMEM((BLK, D), jnp.float32),
        pltpu.VMEM((BLK, D), jnp.float32),
        pltpu.VMEM((BLK,), jnp.int32),
        pltpu.VMEM((BLK,), jnp.int32),
        pltpu.VMEM((BLK,), jnp.float32),
        pltpu.VMEM((BLK,), jnp.int32),
        pltpu.VMEM((BLK,), jnp.int32),
        pltpu.VMEM((BLK,), jnp.float32),
        pltpu.SemaphoreType.DMA,
        pltpu.SemaphoreType.DMA,
        pltpu.SemaphoreType.DMA,
        pltpu.SemaphoreType.DMA,
    ],
)


def _sc_sw_body(dst_hbm, w_hbm, sw_out, acc_sh, wrep_v, dst_v, w_v):
  c = lax.axis_index("c")
  s = lax.axis_index("s")
  wid = c * NS + s

  def zero_rows(i, _):
    for j in range(D // LANES):
      wrep_v[i, pl.ds(j * LANES, LANES)] = jnp.zeros((LANES,), jnp.float32)
    return 0
  lax.fori_loop(0, SWBLK, zero_rows, 0)
  for k in range(ROWS_PER_TILE // SWBLK):
    pltpu.sync_copy(wrep_v,
                    acc_sh.at[pl.ds(s * ROWS_PER_TILE + k * SWBLK, SWBLK)])
  plsc.subcore_barrier()

  base_e = wid * EPT

  def edge_block(blk, _):
    off = base_e + blk * SWBLK
    pltpu.sync_copy(dst_hbm.at[pl.ds(off, SWBLK)], dst_v)
    pltpu.sync_copy(w_hbm.at[pl.ds(off, SWBLK)], w_v)

    def fill_group(g, _):
      wv = w_v[pl.ds(g * LANES, LANES)]
      for k in range(LANES):
        wrep_v[g * LANES + k, pl.ds(0, LANES)] = (
            jnp.broadcast_to(wv[k], (LANES,)))
      return 0
    lax.fori_loop(0, SWBLK // LANES, fill_group, 0)

    pltpu.sync_copy(wrep_v, acc_sh.at[dst_v], add=True)
    return 0

  lax.fori_loop(0, SWNBLK, edge_block, 0)
  plsc.subcore_barrier()
  pltpu.sync_copy(
      acc_sh.at[pl.ds(s * ROWS_PER_TILE, ROWS_PER_TILE)],
      sw_out.at[pl.ds(c * N_PAD + s * ROWS_PER_TILE, ROWS_PER_TILE)])


_sc_sw = pl.kernel(
    _sc_sw_body,
    out_type=jax.ShapeDtypeStruct((NC * N_PAD, D), jnp.float32),
    mesh=plsc.VectorSubcoreMesh(
        core_axis_name="c", subcore_axis_name="s",
        num_cores=NC, num_subcores=NS),
    scratch_types=[
        pltpu.VMEM_SHARED((N_PAD, D), jnp.float32),
        pltpu.VMEM((SWBLK, D), jnp.float32),
        pltpu.VMEM((SWBLK,), jnp.int32),
        pltpu.VMEM((SWBLK,), jnp.float32),
    ],
)


R_BLK = 512


def _dense_body(with_head, a0_ref, a1_ref, h_ref, sw0_ref, sw1_ref,
                wn_ref, wr_ref, br_ref, g_ref, b_ref, fcw_ref, fcb_ref,
                out_ref, logit_ref):
  sw = jnp.maximum(sw0_ref[:, 0:1] + sw1_ref[:, 0:1], 1.0)
  agg = (a0_ref[...] + a1_ref[...]) / sw
  h = h_ref[...]
  out = (jnp.dot(agg, wn_ref[...], preferred_element_type=jnp.float32,
                 precision=lax.Precision.HIGHEST)
         + jnp.dot(h, wr_ref[...], preferred_element_type=jnp.float32,
                   precision=lax.Precision.HIGHEST)
         + br_ref[...])
  out = jnp.maximum(out, 0.0)
  mu = jnp.mean(out, axis=-1, keepdims=True)
  cen = out - mu
  var = jnp.mean(cen * cen, axis=-1, keepdims=True)
  out = cen * lax.rsqrt(var + 1e-5) * g_ref[...] + b_ref[...]
  out_ref[...] = out
  if with_head:
    logit_ref[...] = jnp.dot(
        out, fcw_ref[...], preferred_element_type=jnp.float32,
        precision=lax.Precision.HIGHEST) + fcb_ref[...]


def _make_dense(with_head):
  grid = N_PAD // R_BLK
  row_spec = pl.BlockSpec((R_BLK, D), lambda i: (i, 0))
  full_spec = pl.BlockSpec((D, D), lambda i: (0, 0))
  vec_spec = pl.BlockSpec((1, D), lambda i: (0, 0))
  out_shapes = (jax.ShapeDtypeStruct((N_PAD, D), jnp.float32),
                jax.ShapeDtypeStruct((N_PAD, 1), jnp.float32))
  out_specs = (row_spec, pl.BlockSpec((R_BLK, 1), lambda i: (i, 0)))
  return pl.pallas_call(
      functools.partial(_dense_body, with_head),
      grid=(grid,),
      in_specs=[
          row_spec, row_spec, row_spec, row_spec, row_spec,
          full_spec, full_spec, vec_spec, vec_spec, vec_spec,
          pl.BlockSpec((D, 1), lambda i: (0, 0)),
          pl.BlockSpec((1, 1), lambda i: (0, 0)),
      ],
      out_specs=out_specs,
      out_shape=out_shapes,
  )


_dense_mid = _make_dense(False)
_dense_head = _make_dense(True)


def kernel(x, edge_index, edge_weight,
           W_neigh0, W_root0, b_root0, ln_g0, ln_b0,
           W_neigh1, W_root1, b_root1, ln_g1, ln_b1,
           W_neigh2, W_root2, b_root2, ln_g2, ln_b2,
           fc_w, fc_b):
  h = jnp.zeros((N_PAD, D), jnp.float32).at[:N].set(x)
  pad_e = E_PAD - E
  src = jnp.concatenate([edge_index[0], jnp.zeros((pad_e,), jnp.int32)])
  dst = jnp.concatenate([edge_index[1], jnp.zeros((pad_e,), jnp.int32)])
  w = jnp.concatenate([edge_weight, jnp.zeros((pad_e,), jnp.float32)])

  sw_flat = _sc_sw(dst, w)
  sw0 = sw_flat[:N_PAD]
  sw1 = sw_flat[N_PAD:]

  layers = [
      (W_neigh0, W_root0, b_root0, ln_g0, ln_b0),
      (W_neigh1, W_root1, b_root1, ln_g1, ln_b1),
      (W_neigh2, W_root2, b_root2, ln_g2, ln_b2),
  ]
  logits = None
  for li, (Wn, Wr, br, g, b) in enumerate(layers):
    agg_flat = _sc_agg(h, src, dst, w)
    a0 = agg_flat[:N_PAD]
    a1 = agg_flat[N_PAD:]
    dense = _dense_head if li == 2 else _dense_mid
    h, lg = dense(a0, a1, h, sw0, sw1, Wn.T, Wr.T, br.reshape(1, D),
                  g.reshape(1, D), b.reshape(1, D), fc_w.T,
                  fc_b.reshape(1, 1))
    if li == 2:
      logits = lg
  return (logits[:N, 0], h[:N])
